# baseline (speedup 1.0000x reference)
import math

import jax
import jax.numpy as jnp
from jax.experimental import pallas as pl
from jax.experimental.pallas import tpu as pltpu

_F = 26
_C = 1000
_D = 16
_K = _F * _C

_NSTREAM = 4
_BM = 32


def _mm_body(*refs):
    z_refs = refs[:_NSTREAM]
    w_ref = refs[_NSTREAM]
    o_ref = refs[_NSTREAM + 1]
    for t in range(_NSTREAM):
        o_ref[pl.ds(t * _BM, _BM), :] = jnp.dot(
            z_refs[t][:], w_ref[:], preferred_element_type=jnp.float32
        )


def kernel(z, codebook):
    batch_shape = z.shape[:-1]
    m = math.prod(batch_shape)
    z2 = z.reshape(m, _K)
    w = codebook.reshape(_K, _D)

    rows_per_step = _NSTREAM * _BM
    nsteps = m // rows_per_step

    def make_map(t):
        return lambda i: (i * _NSTREAM + t, 0)

    in_specs = [
        pl.BlockSpec((_BM, _K), make_map(t)) for t in range(_NSTREAM)
    ] + [pl.BlockSpec((_K, _D), lambda i: (0, 0))]

    out = pl.pallas_call(
        _mm_body,
        grid=(nsteps,),
        in_specs=in_specs,
        out_specs=pl.BlockSpec((rows_per_step, _D), lambda i: (i, 0)),
        out_shape=jax.ShapeDtypeStruct((m, _D), jnp.float32),
        compiler_params=pltpu.CompilerParams(
            dimension_semantics=("parallel",)
        ),
    )(*([z2] * _NSTREAM), w)
    return out.reshape(*batch_shape, _D)


# speedup vs baseline: 1.0032x; 1.0032x over previous
import math

import jax
import jax.numpy as jnp
from jax.experimental import pallas as pl
from jax.experimental.pallas import tpu as pltpu

_F = 26
_C = 1000
_D = 16
_K = _F * _C

_KALN = 25984
_CHUNK = 896
_NCH = _KALN // _CHUNK
_NBUF = 4


def _mm_body(z_hbm, w_ref, tail_ref, wtail_ref, o_ref, buf, sems):
    i = pl.program_id(0)

    def copy(c, slot):
        return pltpu.make_async_copy(
            z_hbm.at[:, pl.ds(c * _CHUNK, _CHUNK)],
            buf.at[slot],
            sems.at[slot],
        )

    @pl.when(i == 0)
    def _warmup():
        for s in range(_NBUF - 1):
            copy(s, s).start()

    nxt = i + _NBUF - 1

    @pl.when(nxt < _NCH)
    def _prefetch():
        copy(nxt, jax.lax.rem(nxt, _NBUF)).start()

    slot = jax.lax.rem(i, _NBUF)
    copy(i, slot).wait()

    part = jnp.dot(
        buf[slot],
        w_ref[pl.ds(i * _CHUNK, _CHUNK), :],
        preferred_element_type=jnp.float32,
    )

    @pl.when(i == 0)
    def _init():
        o_ref[:] = part + jnp.dot(
            tail_ref[:], wtail_ref[:], preferred_element_type=jnp.float32
        )

    @pl.when(i > 0)
    def _acc():
        o_ref[:] += part


def kernel(z, codebook):
    batch_shape = z.shape[:-1]
    m = math.prod(batch_shape)
    z2 = z.reshape(m, _K)
    w = codebook.reshape(_K, _D)
    z_tail = z2[:, _KALN:]
    w_tail = w[_KALN:, :]

    out = pl.pallas_call(
        _mm_body,
        grid=(_NCH,),
        in_specs=[
            pl.BlockSpec(memory_space=pltpu.MemorySpace.HBM),
            pl.BlockSpec((_K, _D), lambda i: (0, 0)),
            pl.BlockSpec((m, _K - _KALN), lambda i: (0, 0)),
            pl.BlockSpec((_K - _KALN, _D), lambda i: (0, 0)),
        ],
        out_specs=pl.BlockSpec((m, _D), lambda i: (0, 0)),
        out_shape=jax.ShapeDtypeStruct((m, _D), jnp.float32),
        scratch_shapes=[
            pltpu.VMEM((_NBUF, m, _CHUNK), jnp.float32),
            pltpu.SemaphoreType.DMA((_NBUF,)),
        ],
    )(z2, w, z_tail, w_tail)
    return out.reshape(*batch_shape, _D)
